# baseline (speedup 1.0000x reference)
import functools

import jax
import jax.numpy as jnp
from jax import lax
from jax.experimental import pallas as pl
from jax.experimental.pallas import tpu as pltpu
from jax.experimental.pallas import tpu_sc as plsc

_NUM_CORES = 2
_NUM_SUBCORES = 16
_NUM_WORKERS = _NUM_CORES * _NUM_SUBCORES

_CHUNK = 800

_PACK_ROWS = 50000
_UNPACK_ROWS = 25600


def _widen_block(t_ref, o_ref):
    y = pltpu.bitcast(t_ref[...], jnp.int32)
    even = y & 0xFFFF
    odd = (y >> 16) & 0xFFFF
    o_ref[...] = jnp.concatenate([even, odd], axis=1)


def _narrow_block(g_ref, o_ref):
    g = g_ref[...]
    y = (g[:, 0:64] & 0xFFFF) | (g[:, 64:128] << 16)
    o_ref[...] = pltpu.bitcast(y, jnp.bfloat16)


def _widen_table(table):
    v, d = table.shape
    assert d == 64 and v % _PACK_ROWS == 0
    return pl.pallas_call(
        _widen_block,
        grid=(v // _PACK_ROWS,),
        in_specs=[pl.BlockSpec((_PACK_ROWS, d), lambda i: (i, 0))],
        out_specs=pl.BlockSpec((_PACK_ROWS // 2, 128), lambda i: (i, 0)),
        out_shape=jax.ShapeDtypeStruct((v // 2, 128), jnp.int32),
    )(table)


def _narrow_out_half(g128, n_rows, half, prev=None):
    m = g128.shape[0]
    assert m % _UNPACK_ROWS == 0
    nblk = m // _UNPACK_ROWS

    def body(g_ref, *rest):
        _narrow_block(g_ref, rest[-1])

    in_specs = [pl.BlockSpec((_UNPACK_ROWS, 128), lambda i: (i, 0))]
    operands = [g128]
    kwargs = {}
    if prev is not None:
        in_specs.append(pl.BlockSpec(memory_space=pltpu.MemorySpace.HBM))
        operands.append(prev)
        kwargs["input_output_aliases"] = {1: 0}
    return pl.pallas_call(
        body,
        grid=(nblk,),
        in_specs=in_specs,
        out_specs=pl.BlockSpec((2 * _UNPACK_ROWS, 64),
                               lambda i, h=half, n=nblk: (h * n + i, 0)),
        out_shape=jax.ShapeDtypeStruct((n_rows, 64), jnp.bfloat16),
        **kwargs,
    )(*operands)


def _sc_gather(idx_flat, t32, n_per_w):
    n_iters = n_per_w // _CHUNK
    n2 = n_iters // 2
    n = idx_flat.shape[0]
    w = t32.shape[1]
    c = _CHUNK

    mesh = plsc.VectorSubcoreMesh(
        core_axis_name="c", subcore_axis_name="s",
        num_cores=_NUM_CORES, num_subcores=_NUM_SUBCORES)

    @functools.partial(
        pl.kernel,
        out_type=jax.ShapeDtypeStruct((n, w), jnp.int32),
        mesh=mesh,
        scratch_types=[
            pltpu.VMEM((c,), jnp.int32),
            pltpu.VMEM((c,), jnp.int32),
            pltpu.VMEM((c, w), jnp.int32),
            pltpu.VMEM((c, w), jnp.int32),
            pltpu.SemaphoreType.DMA,
            pltpu.SemaphoreType.DMA,
            pltpu.SemaphoreType.DMA,
            pltpu.SemaphoreType.DMA,
        ],
        compiler_params=pltpu.CompilerParams(use_tc_tiling_on_sc=False),
    )
    def emb(idx_hbm, tab_hbm, out_hbm, idx0, idx1, rows0, rows1,
            sg0, sg1, so0, so1):
        wid = lax.axis_index("s") * _NUM_CORES + lax.axis_index("c")
        base = wid * n_per_w

        pltpu.sync_copy(idx_hbm.at[pl.ds(base, c)], idx0)
        pltpu.async_copy(tab_hbm.at[idx0], rows0, sg0)

        def body(t2, _):
            o0 = base + (2 * t2) * c
            o1 = o0 + c
            o2 = o0 + 2 * c
            pltpu.sync_copy(idx_hbm.at[pl.ds(o1, c)], idx1)

            @pl.when(t2 > 0)
            def _():
                pltpu.make_async_copy(rows1, out_hbm.at[pl.ds(o1 - 2 * c, c)],
                                      so1).wait()

            pltpu.async_copy(tab_hbm.at[idx1], rows1, sg1)
            pltpu.make_async_copy(tab_hbm.at[idx0], rows0, sg0).wait()
            pltpu.async_copy(rows0, out_hbm.at[pl.ds(o0, c)], so0)
            pltpu.make_async_copy(tab_hbm.at[idx1], rows1, sg1).wait()
            pltpu.async_copy(rows1, out_hbm.at[pl.ds(o1, c)], so1)

            @pl.when(t2 + 1 < n2)
            def _():
                pltpu.sync_copy(idx_hbm.at[pl.ds(o2, c)], idx0)
                pltpu.make_async_copy(rows0, out_hbm.at[pl.ds(o0, c)],
                                      so0).wait()
                pltpu.async_copy(tab_hbm.at[idx0], rows0, sg0)

            return 0

        lax.fori_loop(0, n2, body, 0)
        pltpu.make_async_copy(rows0, out_hbm.at[pl.ds(base, c)], so0).wait()
        pltpu.make_async_copy(rows1, out_hbm.at[pl.ds(base, c)], so1).wait()

    return emb(idx_flat, t32)


def kernel(input_ids, table):
    b, s = input_ids.shape
    n = b * s
    v, d = table.shape
    assert n % (_NUM_WORKERS * _CHUNK) == 0

    t128 = _widen_table(table)
    t32 = t128.reshape(v, d)
    nh = n // 2
    idx_a = input_ids[: b // 2].reshape(nh)
    idx_b = input_ids[b // 2 :].reshape(nh)
    g_a = _sc_gather(idx_a, t32, nh // _NUM_WORKERS)
    g_b = _sc_gather(idx_b, t32, nh // _NUM_WORKERS)
    out_a = _narrow_out_half(g_a.reshape(nh // 2, 128), n, 0)
    out = _narrow_out_half(g_b.reshape(nh // 2, 128), n, 1, prev=out_a)
    return out.reshape(b, s, d)


# speedup vs baseline: 1.0048x; 1.0048x over previous
import functools

import jax
import jax.numpy as jnp
from jax import lax
from jax.experimental import pallas as pl
from jax.experimental.pallas import tpu as pltpu
from jax.experimental.pallas import tpu_sc as plsc

_NUM_CORES = 2
_NUM_SUBCORES = 16
_NUM_WORKERS = _NUM_CORES * _NUM_SUBCORES

_CHUNK = 800

_PACK_ROWS = 50000
_UNPACK_ROWS = 25600


def _widen_block(t_ref, o_ref):
    y = pltpu.bitcast(t_ref[...], jnp.int32)
    even = y & 0xFFFF
    odd = (y >> 16) & 0xFFFF
    o_ref[...] = jnp.concatenate([even, odd], axis=1)


def _narrow_block(g_ref, o_ref):
    g = g_ref[...]
    y = (g[:, 0:64] & 0xFFFF) | (g[:, 64:128] << 16)
    o_ref[...] = pltpu.bitcast(y, jnp.bfloat16)


def _widen_table(table):
    v, d = table.shape
    assert d == 64 and v % _PACK_ROWS == 0
    return pl.pallas_call(
        _widen_block,
        grid=(v // _PACK_ROWS,),
        in_specs=[pl.BlockSpec((_PACK_ROWS, d), lambda i: (i, 0))],
        out_specs=pl.BlockSpec((_PACK_ROWS // 2, 128), lambda i: (i, 0)),
        out_shape=jax.ShapeDtypeStruct((v // 2, 128), jnp.int32),
    )(table)


def _narrow_out(g128):
    m = g128.shape[0]
    assert m % _UNPACK_ROWS == 0
    return pl.pallas_call(
        _narrow_block,
        grid=(m // _UNPACK_ROWS,),
        in_specs=[pl.BlockSpec((_UNPACK_ROWS, 128), lambda i: (i, 0))],
        out_specs=pl.BlockSpec((2 * _UNPACK_ROWS, 64), lambda i: (i, 0)),
        out_shape=jax.ShapeDtypeStruct((2 * m, 64), jnp.bfloat16),
    )(g128)


def _sc_gather(idx_flat, t32, n_per_w):
    n_iters = n_per_w // _CHUNK
    n2 = n_iters // 2
    n = idx_flat.shape[0]
    w = t32.shape[1]
    c = _CHUNK

    mesh = plsc.VectorSubcoreMesh(
        core_axis_name="c", subcore_axis_name="s",
        num_cores=_NUM_CORES, num_subcores=_NUM_SUBCORES)

    @functools.partial(
        pl.kernel,
        out_type=jax.ShapeDtypeStruct((n, w), jnp.int32),
        mesh=mesh,
        scratch_types=[
            pltpu.VMEM((c,), jnp.int32),
            pltpu.VMEM((c,), jnp.int32),
            pltpu.VMEM((c, w), jnp.int32),
            pltpu.VMEM((c, w), jnp.int32),
            pltpu.SemaphoreType.DMA,
            pltpu.SemaphoreType.DMA,
            pltpu.SemaphoreType.DMA,
            pltpu.SemaphoreType.DMA,
        ],
        compiler_params=pltpu.CompilerParams(use_tc_tiling_on_sc=False),
    )
    def emb(idx_hbm, tab_hbm, out_hbm, idx0, idx1, rows0, rows1,
            sg0, sg1, so0, so1):
        wid = lax.axis_index("s") * _NUM_CORES + lax.axis_index("c")
        base = wid * n_per_w

        pltpu.sync_copy(idx_hbm.at[pl.ds(base, c)], idx0)
        pltpu.async_copy(tab_hbm.at[idx0], rows0, sg0)

        def body(t2, _):
            o0 = base + (2 * t2) * c
            o1 = o0 + c
            o2 = o0 + 2 * c
            pltpu.sync_copy(idx_hbm.at[pl.ds(o1, c)], idx1)

            @pl.when(t2 > 0)
            def _():
                pltpu.make_async_copy(rows1, out_hbm.at[pl.ds(o1 - 2 * c, c)],
                                      so1).wait()

            pltpu.async_copy(tab_hbm.at[idx1], rows1, sg1)
            pltpu.make_async_copy(tab_hbm.at[idx0], rows0, sg0).wait()
            pltpu.async_copy(rows0, out_hbm.at[pl.ds(o0, c)], so0)
            pltpu.make_async_copy(tab_hbm.at[idx1], rows1, sg1).wait()
            pltpu.async_copy(rows1, out_hbm.at[pl.ds(o1, c)], so1)

            @pl.when(t2 + 1 < n2)
            def _():
                pltpu.sync_copy(idx_hbm.at[pl.ds(o2, c)], idx0)
                pltpu.make_async_copy(rows0, out_hbm.at[pl.ds(o0, c)],
                                      so0).wait()
                pltpu.async_copy(tab_hbm.at[idx0], rows0, sg0)

            return 0

        lax.fori_loop(0, n2, body, 0)
        pltpu.make_async_copy(rows0, out_hbm.at[pl.ds(base, c)], so0).wait()
        pltpu.make_async_copy(rows1, out_hbm.at[pl.ds(base, c)], so1).wait()

    return emb(idx_flat, t32)


def kernel(input_ids, table):
    b, s = input_ids.shape
    n = b * s
    v, d = table.shape
    assert n % (_NUM_WORKERS * _CHUNK) == 0

    t128 = _widen_table(table)
    t32 = t128.reshape(v, d)
    idx_flat = input_ids.reshape(n)
    g32 = _sc_gather(idx_flat, t32, n // _NUM_WORKERS)
    g128 = g32.reshape(n // 2, 128)
    out = _narrow_out(g128)
    return out.reshape(b, s, d)
